# baseline (speedup 1.0000x reference)
import jax
import jax.numpy as jnp
from jax.experimental import pallas as pl
from jax.experimental.pallas import tpu as pltpu

_N = 4096
_D = 128
_H = 64
_Z = 16
_CAUSAL = 8
_E = 131072
_M = 2
_SIZE_COEF = 1e-4
_KL_COEF = 1.0
_VGAE_COEF = 1.0

_RT = 128
_G = _N // _RT
_UK = _N + 2 * _E
_UT = _UK // _G


def _sweep_body(z_ref, zt_ref, ct_ref, mu_ref, lv_ref, zi_ref, zj_ref, out_ref):
    z = z_ref[...]
    s = jnp.dot(z, zt_ref[...], preferred_element_type=jnp.float32)
    r = jax.nn.sigmoid(s)
    bce_all = jnp.sum(jnp.log(1.0 - r + 1e-8))
    c = z[:, :_CAUSAL]
    sc = jnp.dot(c, ct_ref[...], preferred_element_type=jnp.float32)
    size_sum = jnp.sum(jax.nn.sigmoid(sc))
    zi = zi_ref[...]
    zj = zj_ref[...]
    du = jnp.sum(zi * zj, axis=1)
    ru = jax.nn.sigmoid(du)
    corr = jnp.sum(jnp.log(ru + 1e-8) - jnp.log(1.0 - ru + 1e-8))
    muv = mu_ref[...]
    lvv = lv_ref[...]
    klz_part = jnp.sum(1.0 + lvv - muv * muv - jnp.exp(lvv))
    row = jnp.concatenate([
        jnp.reshape(bce_all + corr, (1,)),
        jnp.reshape(size_sum, (1,)),
        jnp.reshape(klz_part, (1,)),
        jnp.zeros((125,), jnp.float32),
    ])[None, :]
    out_ref[pl.ds(pl.program_id(0), 1), :] = row


def _decoder_losses(all_z, mu, logvar, zi_rows, zj_rows):
    zt = all_z.T
    ct = all_z[:, :_CAUSAL].T
    out = pl.pallas_call(
        _sweep_body,
        grid=(_G,),
        in_specs=[
            pl.BlockSpec((_RT, _Z), lambda i: (i, 0)),
            pl.BlockSpec((_Z, _N), lambda i: (0, 0)),
            pl.BlockSpec((_CAUSAL, _N), lambda i: (0, 0)),
            pl.BlockSpec((_RT, _Z), lambda i: (i, 0)),
            pl.BlockSpec((_RT, _Z), lambda i: (i, 0)),
            pl.BlockSpec((_UT, _Z), lambda i: (i, 0)),
            pl.BlockSpec((_UT, _Z), lambda i: (i, 0)),
        ],
        out_specs=pl.BlockSpec((_G, 128), lambda i: (0, 0)),
        out_shape=jax.ShapeDtypeStruct((_G, 128), jnp.float32),
    )(all_z, zt, ct, mu, logvar, zi_rows, zj_rows)
    sums = jnp.sum(out, axis=0)
    bce_sum, size_sum, klz_sum = sums[0], sums[1], sums[2]
    bce = -bce_sum / (_N * _N)
    size_loss = _SIZE_COEF * size_sum / (_N * _N)
    klz = (-0.5 / _N) * klz_sum
    return bce, size_loss, klz


def _deg(dst):
    d = jax.ops.segment_sum(jnp.ones((_E, 1), jnp.float32), dst, num_segments=_N)
    return jnp.clip(d, 1.0)


def _gcn_relu(x, src, dst, W, deg, edge_w=None):
    h = x @ W
    m = jnp.take(h, src, axis=0)
    if edge_w is not None:
        m = m * edge_w
    agg = jax.ops.segment_sum(m, dst, num_segments=_N)
    return jax.nn.relu(agg / deg + h)


def _gcn_lin(x, src, dst, W, deg):
    h = x @ W
    agg = jax.ops.segment_sum(jnp.take(h, src, axis=0), dst, num_segments=_N)
    return agg / deg + h


def _clf(x, src, dst, Wc1, Wc2, Wout, deg, edge_w=None):
    h = _gcn_relu(x, src, dst, Wc1, deg, edge_w)
    h = _gcn_relu(h, src, dst, Wc2, deg, edge_w)
    g = jnp.mean(h, axis=0, keepdims=True)
    return g @ Wout


def kernel(x, edge_index, eps, W1, W_mu, W_lv, Wc1, Wc2, Wout):
    src = edge_index[0]
    dst = edge_index[1]
    deg = _deg(dst)

    orig_logits = _clf(x, src, dst, Wc1, Wc2, Wout, deg)

    h = _gcn_relu(x, src, dst, W1, deg)
    mu = _gcn_lin(h, src, dst, W_mu, deg)
    logvar = _gcn_lin(h, src, dst, W_lv, deg)
    all_z = mu + jnp.exp(0.5 * logvar) * eps
    caul_z = all_z[:, :_CAUSAL]

    c_src = jnp.take(caul_z, src, axis=0)
    c_dst = jnp.take(caul_z, dst, axis=0)
    edge_attn = jax.nn.sigmoid(jnp.sum(c_src * c_dst, axis=1, keepdims=True))

    masked_logits = _clf(x, src, dst, Wc1, Wc2, Wout, deg, edge_w=edge_attn)

    keys = jnp.concatenate([
        jnp.arange(_N, dtype=jnp.int32) * (_N + 1),
        src * _N + dst,
        dst * _N + src,
    ])
    skeys = jnp.sort(keys)
    uniq = jnp.concatenate([jnp.ones((1,), jnp.bool_), skeys[1:] != skeys[:-1]])
    ui = skeys // _N
    uj = skeys % _N
    zi_rows = jnp.where(uniq[:, None], jnp.take(all_z, ui, axis=0), 0.0)
    zj_rows = jnp.take(all_z, uj, axis=0)

    bce, size_loss, klz = _decoder_losses(all_z, mu, logvar, zi_rows, zj_rows)

    logp = jax.nn.log_softmax(masked_logits, axis=1)
    p = jax.nn.softmax(orig_logits, axis=1)
    kl_loss = _KL_COEF * jnp.sum(p * (jnp.log(p + 1e-12) - logp)) / masked_logits.shape[0]
    vgae_loss = _VGAE_COEF * (bce + klz)
    loss = size_loss + kl_loss + vgae_loss
    return (loss, orig_logits, edge_attn.reshape(-1))


# speedup vs baseline: 1.4833x; 1.4833x over previous
import jax
import jax.numpy as jnp
from jax.experimental import pallas as pl
from jax.experimental.pallas import tpu as pltpu

_N = 4096
_D = 128
_H = 64
_Z = 16
_CAUSAL = 8
_E = 131072
_M = 2
_SIZE_COEF = 1e-4
_KL_COEF = 1.0
_VGAE_COEF = 1.0

_RT = 128
_G = _N // _RT


def _sweep_body(z_ref, zt_ref, ct_ref, mu_ref, lv_ref, org_ref, out_ref):
    i = pl.program_id(0)
    z = z_ref[...]
    s = jnp.dot(z, zt_ref[...], preferred_element_type=jnp.float32)
    r = jax.nn.sigmoid(s)
    log_neg = jnp.log(1.0 - r + 1e-8)
    bce_all = jnp.sum(log_neg)
    rows = jax.lax.broadcasted_iota(jnp.int32, (_RT, _N), 0) + i * _RT
    cols = jax.lax.broadcasted_iota(jnp.int32, (_RT, _N), 1)
    on = jnp.logical_or(org_ref[...] > 0.0, rows == cols)
    corr = jnp.sum(jnp.where(on, jnp.log(r + 1e-8) - log_neg, 0.0))
    c = z[:, :_CAUSAL]
    sc = jnp.dot(c, ct_ref[...], preferred_element_type=jnp.float32)
    size_sum = jnp.sum(jax.nn.sigmoid(sc))
    muv = mu_ref[...]
    lvv = lv_ref[...]
    klz_part = jnp.sum(1.0 + lvv - muv * muv - jnp.exp(lvv))
    row = jnp.concatenate([
        jnp.reshape(bce_all + corr, (1,)),
        jnp.reshape(size_sum, (1,)),
        jnp.reshape(klz_part, (1,)),
        jnp.zeros((125,), jnp.float32),
    ])[None, :]
    out_ref[pl.ds(i, 1), :] = row


def _decoder_losses(all_z, mu, logvar, org_mask):
    zt = all_z.T
    ct = all_z[:, :_CAUSAL].T
    out = pl.pallas_call(
        _sweep_body,
        grid=(_G,),
        in_specs=[
            pl.BlockSpec((_RT, _Z), lambda i: (i, 0)),
            pl.BlockSpec((_Z, _N), lambda i: (0, 0)),
            pl.BlockSpec((_CAUSAL, _N), lambda i: (0, 0)),
            pl.BlockSpec((_RT, _Z), lambda i: (i, 0)),
            pl.BlockSpec((_RT, _Z), lambda i: (i, 0)),
            pl.BlockSpec((_RT, _N), lambda i: (i, 0)),
        ],
        out_specs=pl.BlockSpec((_G, 128), lambda i: (0, 0)),
        out_shape=jax.ShapeDtypeStruct((_G, 128), jnp.float32),
    )(all_z, zt, ct, mu, logvar, org_mask)
    sums = jnp.sum(out, axis=0)
    bce = -sums[0] / (_N * _N)
    size_loss = _SIZE_COEF * sums[1] / (_N * _N)
    klz = (-0.5 / _N) * sums[2]
    return bce, size_loss, klz


def kernel(x, edge_index, eps, W1, W_mu, W_lv, Wc1, Wc2, Wout):
    src = edge_index[0]
    dst = edge_index[1]

    x_aug = jnp.concatenate([x, jnp.ones((_N, 1), jnp.float32)], axis=1)
    s_xa = jax.ops.segment_sum(jnp.take(x_aug, src, axis=0), dst, num_segments=_N)
    s_x = s_xa[:, :_D]
    deg = jnp.clip(s_xa[:, _D:], 1.0)

    hx1 = x @ Wc1
    h1o = jax.nn.relu((s_x @ Wc1) / deg + hx1)
    s1o = jax.ops.segment_sum(jnp.take(h1o, src, axis=0), dst, num_segments=_N)
    h2o = jax.nn.relu((s1o @ Wc2) / deg + h1o @ Wc2)
    orig_logits = jnp.mean(h2o, axis=0, keepdims=True) @ Wout

    h = jax.nn.relu((s_x @ W1) / deg + x @ W1)
    s_h = jax.ops.segment_sum(jnp.take(h, src, axis=0), dst, num_segments=_N)
    mu = (s_h @ W_mu) / deg + h @ W_mu
    logvar = (s_h @ W_lv) / deg + h @ W_lv
    all_z = mu + jnp.exp(0.5 * logvar) * eps
    caul_z = all_z[:, :_CAUSAL]

    c_src = jnp.take(caul_z, src, axis=0)
    c_dst = jnp.take(caul_z, dst, axis=0)
    edge_attn = jax.nn.sigmoid(jnp.sum(c_src * c_dst, axis=1, keepdims=True))

    s_xw = jax.ops.segment_sum(jnp.take(x, src, axis=0) * edge_attn, dst,
                               num_segments=_N)
    h1m = jax.nn.relu((s_xw @ Wc1) / deg + hx1)
    s_m = jax.ops.segment_sum(jnp.take(h1m, src, axis=0) * edge_attn, dst,
                              num_segments=_N)
    h2m = jax.nn.relu((s_m @ Wc2) / deg + h1m @ Wc2)
    masked_logits = jnp.mean(h2m, axis=0, keepdims=True) @ Wout

    org_mask = (jnp.zeros((_N, _N), jnp.float32)
                .at[src, dst].set(1.0)
                .at[dst, src].set(1.0))

    bce, size_loss, klz = _decoder_losses(all_z, mu, logvar, org_mask)

    logp = jax.nn.log_softmax(masked_logits, axis=1)
    p = jax.nn.softmax(orig_logits, axis=1)
    kl_loss = _KL_COEF * jnp.sum(p * (jnp.log(p + 1e-12) - logp)) / masked_logits.shape[0]
    vgae_loss = _VGAE_COEF * (bce + klz)
    loss = size_loss + kl_loss + vgae_loss
    return (loss, orig_logits, edge_attn.reshape(-1))
